# baseline (speedup 1.0000x reference)
import dataclasses
import functools

import jax
import jax.numpy as jnp
from jax import lax
from jax.experimental import pallas as pl
from jax.experimental.pallas import tpu as pltpu
from jax.experimental.pallas import tpu_sc as plsc

_N = 10000
_E = 320000
_GD = 128
_H1 = 256
_H2 = 256

_NC = 2
_NS = 16
_L = 16

_RPS = 632
_NPAD = _RPS * _NS

_D2 = 3

_mesh = plsc.VectorSubcoreMesh(core_axis_name="c", subcore_axis_name="s")

_sc_params = pltpu.CompilerParams()
if "needs_layout_passes" in pltpu.CompilerParams.__dataclass_fields__:
    _sc_params = dataclasses.replace(_sc_params, needs_layout_passes=False)


def _zero_acc(rows_v, acc_sh, s):
    bufrows = rows_v.shape[0]
    zv = jnp.zeros((_L,), jnp.float32)

    @pl.loop(0, bufrows)
    def _(r):
        for kk in range(_GD // _L):
            rows_v[r, pl.ds(kk * _L, _L)] = zv

    base_r = s * _RPS
    nfull = _RPS // bufrows
    tail = _RPS - nfull * bufrows
    for t in range(nfull):
        pltpu.sync_copy(rows_v.at[pl.ds(0, bufrows)],
                        acc_sh.at[pl.ds(base_r + t * bufrows, bufrows)])
    if tail:
        pltpu.sync_copy(rows_v.at[pl.ds(0, tail)],
                        acc_sh.at[pl.ds(base_r + nfull * bufrows, tail)])


_D = 3


def _ring(D, nfull, start_fetch, wait_fetch, process, start_scatter,
          wait_scatter):
    for b in range(min(D, nfull)):
        start_fetch(b, b)
    nloop = max(0, (nfull - D) // D) * D

    @pl.loop(0, nloop, step=D)
    def _(j):
        for b in range(D):
            wait_fetch(j + b, b)
            process(b)
            start_scatter(b)
        for b in range(D):
            wait_scatter(b)
            start_fetch(j + D + b, b)

    for jj in range(nloop, nfull):
        b = jj % D
        wait_fetch(jj, b)
        process(b)
        start_scatter(b)
        if jj + D < nfull:
            wait_scatter(b)
            start_fetch(jj + D, b)
    for b in range(min(D, nfull)):
        wait_scatter(b)


def _sc_layer1(src, dst, w, x):
    chunk = 96
    eps = _E // (_NC * _NS)
    nfull = eps // chunk
    main = nfull * chunk
    tail = eps - main

    @functools.partial(
        pl.kernel,
        out_type=jax.ShapeDtypeStruct((_NC, _NPAD, _GD), jnp.float32),
        mesh=_mesh,
        compiler_params=_sc_params,
        scratch_types=[
            pltpu.VMEM((main,), jnp.int32),
            pltpu.VMEM((_D, chunk), jnp.int32),
            pltpu.VMEM((_D, chunk), jnp.float32),
            pltpu.VMEM((tail,), jnp.int32),
            pltpu.VMEM((tail,), jnp.int32),
        ] + [pltpu.VMEM((chunk, _GD), jnp.float32)] * _D
          + [pltpu.VMEM_SHARED((_NPAD, _GD), jnp.float32)]
          + [pltpu.SemaphoreType.DMA] * (2 * _D),
    )
    def k(src_hbm, dst_hbm, w_hbm, x_hbm, out_hbm,
          srcb_v, dstc_v, wcb_v, srct_v, dstt_v, *rest):
        rows = rest[:_D]
        acc_sh = rest[_D]
        gsem = rest[_D + 1:_D + 1 + _D]
        ssem = rest[_D + 1 + _D:]
        c = lax.axis_index("c")
        s = lax.axis_index("s")
        _zero_acc(rows[0], acc_sh, s)

        base_e = (c * _NS + s) * eps
        pltpu.sync_copy(src_hbm.at[pl.ds(base_e, main)], srcb_v)
        plsc.subcore_barrier()

        def start_fetch(jj, b):
            off = base_e + jj * chunk
            pltpu.async_copy(dst_hbm.at[pl.ds(off, chunk)], dstc_v.at[b],
                             gsem[b])
            pltpu.async_copy(w_hbm.at[pl.ds(off, chunk)], wcb_v.at[b],
                             gsem[b])
            pltpu.async_copy(x_hbm.at[srcb_v.at[pl.ds(jj * chunk, chunk)]],
                             rows[b], gsem[b])

        def wait_fetch(jj, b):
            off = base_e + jj * chunk
            pltpu.make_async_copy(dst_hbm.at[pl.ds(off, chunk)],
                                  dstc_v.at[b], gsem[b]).wait()
            pltpu.make_async_copy(w_hbm.at[pl.ds(off, chunk)], wcb_v.at[b],
                                  gsem[b]).wait()
            pltpu.make_async_copy(
                x_hbm.at[srcb_v.at[pl.ds(jj * chunk, chunk)]],
                rows[b], gsem[b]).wait()

        def start_scatter(b):
            pltpu.async_copy(rows[b], acc_sh.at[dstc_v.at[b]], ssem[b],
                             add=True)

        def wait_scatter(b):
            pltpu.make_async_copy(rows[b], acc_sh.at[dstc_v.at[b]],
                                  ssem[b]).wait()

        def scale_rows(rows_v, wv_ref, n):
            @plsc.parallel_loop(0, n, 1, unroll=8,
                                carry=jnp.zeros((_L,), jnp.int32))
            def _(i, isplat):
                wv = plsc.load_gather(wv_ref, [isplat])
                for kk in range(_GD // _L):
                    sl = (i, pl.ds(kk * _L, _L))
                    rows_v[sl] = rows_v[sl] * wv
                return isplat + 1

        def process(b):
            scale_rows(rows[b], wcb_v.at[b], chunk)

        _ring(_D, nfull, start_fetch, wait_fetch, process, start_scatter,
              wait_scatter)

        if tail:
            off = base_e + main
            pltpu.sync_copy(src_hbm.at[pl.ds(off, tail)], srct_v)
            pltpu.sync_copy(dst_hbm.at[pl.ds(off, tail)], dstt_v)
            pltpu.sync_copy(w_hbm.at[pl.ds(off, tail)],
                            wcb_v.at[0, pl.ds(0, tail)])
            pltpu.sync_copy(x_hbm.at[srct_v], rows[0].at[pl.ds(0, tail)])
            scale_rows(rows[0], wcb_v.at[0], tail)
            pltpu.sync_copy(rows[0].at[pl.ds(0, tail)], acc_sh.at[dstt_v],
                            add=True)

        plsc.subcore_barrier()
        base_r = s * _RPS
        pltpu.sync_copy(acc_sh.at[pl.ds(base_r, _RPS)],
                        out_hbm.at[c, pl.ds(base_r, _RPS)])

    return k(src, dst, w, x)


def _sc_layer2(src, dst, hlo, hhi):
    chunk = 64
    eps = _E // _NS
    nfull = eps // chunk
    main = nfull * chunk
    tail = eps - main

    @functools.partial(
        pl.kernel,
        out_type=jax.ShapeDtypeStruct((_NC, _NPAD, _GD), jnp.float32),
        mesh=_mesh,
        compiler_params=_sc_params,
        scratch_types=[
            pltpu.VMEM((main,), jnp.int32),
            pltpu.VMEM((_D2, chunk), jnp.int32),
            pltpu.VMEM((tail,), jnp.int32),
            pltpu.VMEM((tail,), jnp.int32),
        ] + [pltpu.VMEM((chunk, _GD), jnp.float32)] * _D2
          + [pltpu.VMEM_SHARED((_NPAD, _GD), jnp.float32)]
          + [pltpu.SemaphoreType.DMA] * (2 * _D2),
    )
    def k(src_hbm, dst_hbm, hlo_hbm, hhi_hbm, out_hbm,
          srcb_v, dstc_v, srct_v, dstt_v, *rest):
        rows = rest[:_D2]
        acc_sh = rest[_D2]
        gsem = rest[_D2 + 1:_D2 + 1 + _D2]
        ssem = rest[_D2 + 1 + _D2:]
        c = lax.axis_index("c")
        s = lax.axis_index("s")
        _zero_acc(rows[0], acc_sh, s)

        base_e = s * eps
        pltpu.sync_copy(src_hbm.at[pl.ds(base_e, main)], srcb_v)
        plsc.subcore_barrier()

        def run(table):
            def start_fetch(jj, b):
                off = base_e + jj * chunk
                pltpu.async_copy(dst_hbm.at[pl.ds(off, chunk)],
                                 dstc_v.at[b], gsem[b])
                pltpu.async_copy(
                    table.at[srcb_v.at[pl.ds(jj * chunk, chunk)]],
                    rows[b], gsem[b])

            def wait_fetch(jj, b):
                off = base_e + jj * chunk
                pltpu.make_async_copy(dst_hbm.at[pl.ds(off, chunk)],
                                      dstc_v.at[b], gsem[b]).wait()
                pltpu.make_async_copy(
                    table.at[srcb_v.at[pl.ds(jj * chunk, chunk)]],
                    rows[b], gsem[b]).wait()

            def start_scatter(b):
                pltpu.async_copy(rows[b], acc_sh.at[dstc_v.at[b]], ssem[b],
                                 add=True)

            def wait_scatter(b):
                pltpu.make_async_copy(rows[b], acc_sh.at[dstc_v.at[b]],
                                      ssem[b]).wait()

            _ring(_D2, nfull, start_fetch, wait_fetch, lambda b: None,
                  start_scatter, wait_scatter)

            if tail:
                off = base_e + main
                pltpu.sync_copy(src_hbm.at[pl.ds(off, tail)], srct_v)
                pltpu.sync_copy(dst_hbm.at[pl.ds(off, tail)], dstt_v)
                pltpu.sync_copy(table.at[srct_v], rows[0].at[pl.ds(0, tail)])
                pltpu.sync_copy(rows[0].at[pl.ds(0, tail)], acc_sh.at[dstt_v],
                                add=True)

        @pl.when(c == 0)
        def _():
            run(hlo_hbm)

        @pl.when(c == 1)
        def _():
            run(hhi_hbm)

        plsc.subcore_barrier()
        base_r = s * _RPS
        pltpu.sync_copy(acc_sh.at[pl.ds(base_r, _RPS)],
                        out_hbm.at[c, pl.ds(base_r, _RPS)])

    return k(src, dst, hlo, hhi)


_BN = 1000


def _tc_h1(agg1, x, W_rel, W_root, b):
    def body(a_ref, x_ref, wrel_ref, wroot_ref, b_ref, o_ref, o2_ref):
        aggsum = a_ref[0, :, :] + a_ref[1, :, :]
        r = jnp.dot(aggsum, wrel_ref[...], preferred_element_type=jnp.float32)
        r = r + jnp.dot(x_ref[...], wroot_ref[...],
                        preferred_element_type=jnp.float32) + b_ref[...]
        o_ref[...] = r[:, :_GD]
        o2_ref[...] = r[:, _GD:]

    return pl.pallas_call(
        body,
        grid=(_N // _BN,),
        in_specs=[
            pl.BlockSpec((_NC, _BN, _GD), lambda i: (0, i, 0)),
            pl.BlockSpec((_BN, _GD), lambda i: (i, 0)),
            pl.BlockSpec((_GD, _H1), lambda i: (0, 0)),
            pl.BlockSpec((_GD, _H1), lambda i: (0, 0)),
            pl.BlockSpec((1, _H1), lambda i: (0, 0)),
        ],
        out_specs=[pl.BlockSpec((_BN, _GD), lambda i: (i, 0)),
                   pl.BlockSpec((_BN, _GD), lambda i: (i, 0))],
        out_shape=[jax.ShapeDtypeStruct((_N, _GD), jnp.float32),
                   jax.ShapeDtypeStruct((_N, _GD), jnp.float32)],
    )(agg1, x, W_rel, W_root, b)


def _tc_out(agg2, hlo, hhi, W_rel, W_root, b):
    def body(a_ref, hlo_ref, hhi_ref, wrel_ref, wroot_ref, b_ref, o_ref):
        r = jnp.dot(a_ref[0, :, :], wrel_ref[:_GD, :],
                    preferred_element_type=jnp.float32)
        r = r + jnp.dot(a_ref[1, :, :], wrel_ref[_GD:, :],
                        preferred_element_type=jnp.float32)
        r = r + jnp.dot(hlo_ref[...], wroot_ref[:_GD, :],
                        preferred_element_type=jnp.float32)
        r = r + jnp.dot(hhi_ref[...], wroot_ref[_GD:, :],
                        preferred_element_type=jnp.float32)
        o_ref[...] = r + b_ref[...]

    return pl.pallas_call(
        body,
        grid=(_N // _BN,),
        in_specs=[
            pl.BlockSpec((_NC, _BN, _GD), lambda i: (0, i, 0)),
            pl.BlockSpec((_BN, _GD), lambda i: (i, 0)),
            pl.BlockSpec((_BN, _GD), lambda i: (i, 0)),
            pl.BlockSpec((_H1, _H2), lambda i: (0, 0)),
            pl.BlockSpec((_H1, _H2), lambda i: (0, 0)),
            pl.BlockSpec((1, _H2), lambda i: (0, 0)),
        ],
        out_specs=pl.BlockSpec((_BN, _H2), lambda i: (i, 0)),
        out_shape=jax.ShapeDtypeStruct((_N, _H2), jnp.float32),
    )(agg2, hlo, hhi, W_rel, W_root, b)


def kernel(node_features, edge_index, edge_type, W1_rel, b1_rel, W1_root,
           W2_rel, b2_rel, W2_root):
    src = edge_index[0]
    dst = edge_index[1]
    b1 = b1_rel.reshape(1, _H1)
    b2 = b2_rel.reshape(1, _H2)

    agg1 = _sc_layer1(src, dst, edge_type, node_features)
    hlo, hhi = _tc_h1(agg1, node_features, W1_rel, W1_root, b1)
    agg2 = _sc_layer2(src, dst, hlo, hhi)
    out = _tc_out(agg2, hlo, hhi, W2_rel, W2_root, b2)
    return out


# speedup vs baseline: 1.0509x; 1.0509x over previous
import dataclasses
import functools

import jax
import jax.numpy as jnp
from jax import lax
from jax.experimental import pallas as pl
from jax.experimental.pallas import tpu as pltpu
from jax.experimental.pallas import tpu_sc as plsc

_N = 10000
_E = 320000
_GD = 128
_H1 = 256
_H2 = 256

_NC = 2
_NS = 16
_L = 16

_RPS = 632
_NPAD = _RPS * _NS

_D2 = 4

_mesh = plsc.VectorSubcoreMesh(core_axis_name="c", subcore_axis_name="s")

_sc_params = pltpu.CompilerParams()
if "needs_layout_passes" in pltpu.CompilerParams.__dataclass_fields__:
    _sc_params = dataclasses.replace(_sc_params, needs_layout_passes=False)


def _zero_acc(rows_v, acc_sh, s):
    bufrows = rows_v.shape[0]
    zv = jnp.zeros((_L,), jnp.float32)

    @pl.loop(0, bufrows)
    def _(r):
        for kk in range(_GD // _L):
            rows_v[r, pl.ds(kk * _L, _L)] = zv

    base_r = s * _RPS
    nfull = _RPS // bufrows
    tail = _RPS - nfull * bufrows
    for t in range(nfull):
        pltpu.sync_copy(rows_v.at[pl.ds(0, bufrows)],
                        acc_sh.at[pl.ds(base_r + t * bufrows, bufrows)])
    if tail:
        pltpu.sync_copy(rows_v.at[pl.ds(0, tail)],
                        acc_sh.at[pl.ds(base_r + nfull * bufrows, tail)])


_D = 3


def _ring(D, nfull, start_fetch, wait_fetch, process, start_scatter,
          wait_scatter):
    for b in range(min(D, nfull)):
        start_fetch(b, b)
    nloop = max(0, (nfull - D) // D) * D

    @pl.loop(0, nloop, step=D)
    def _(j):
        for b in range(D):
            wait_fetch(j + b, b)
            process(b)
            start_scatter(b)
        for b in range(D):
            wait_scatter(b)
            start_fetch(j + D + b, b)

    for jj in range(nloop, nfull):
        b = jj % D
        wait_fetch(jj, b)
        process(b)
        start_scatter(b)
        if jj + D < nfull:
            wait_scatter(b)
            start_fetch(jj + D, b)
    for b in range(min(D, nfull)):
        wait_scatter(b)


def _sc_layer1(src, dst, w, x):
    chunk = 96
    eps = _E // (_NC * _NS)
    nfull = eps // chunk
    main = nfull * chunk
    tail = eps - main

    @functools.partial(
        pl.kernel,
        out_type=jax.ShapeDtypeStruct((_NC, _NPAD, _GD), jnp.float32),
        mesh=_mesh,
        compiler_params=_sc_params,
        scratch_types=[
            pltpu.VMEM((main,), jnp.int32),
            pltpu.VMEM((_D, chunk), jnp.int32),
            pltpu.VMEM((_D, chunk), jnp.float32),
            pltpu.VMEM((tail,), jnp.int32),
            pltpu.VMEM((tail,), jnp.int32),
        ] + [pltpu.VMEM((chunk, _GD), jnp.float32)] * _D
          + [pltpu.VMEM_SHARED((_NPAD, _GD), jnp.float32)]
          + [pltpu.SemaphoreType.DMA] * (2 * _D),
    )
    def k(src_hbm, dst_hbm, w_hbm, x_hbm, out_hbm,
          srcb_v, dstc_v, wcb_v, srct_v, dstt_v, *rest):
        rows = rest[:_D]
        acc_sh = rest[_D]
        gsem = rest[_D + 1:_D + 1 + _D]
        ssem = rest[_D + 1 + _D:]
        c = lax.axis_index("c")
        s = lax.axis_index("s")
        base_e = (c * _NS + s) * eps
        srcb_cp = pltpu.async_copy(src_hbm.at[pl.ds(base_e, main)], srcb_v,
                                   gsem[0])
        _zero_acc(rows[0], acc_sh, s)
        srcb_cp.wait()
        plsc.subcore_barrier()

        def start_fetch(jj, b):
            off = base_e + jj * chunk
            pltpu.async_copy(dst_hbm.at[pl.ds(off, chunk)], dstc_v.at[b],
                             gsem[b])
            pltpu.async_copy(w_hbm.at[pl.ds(off, chunk)], wcb_v.at[b],
                             gsem[b])
            pltpu.async_copy(x_hbm.at[srcb_v.at[pl.ds(jj * chunk, chunk)]],
                             rows[b], gsem[b])

        def wait_fetch(jj, b):
            off = base_e + jj * chunk
            pltpu.make_async_copy(dst_hbm.at[pl.ds(off, chunk)],
                                  dstc_v.at[b], gsem[b]).wait()
            pltpu.make_async_copy(w_hbm.at[pl.ds(off, chunk)], wcb_v.at[b],
                                  gsem[b]).wait()
            pltpu.make_async_copy(
                x_hbm.at[srcb_v.at[pl.ds(jj * chunk, chunk)]],
                rows[b], gsem[b]).wait()

        def start_scatter(b):
            pltpu.async_copy(rows[b], acc_sh.at[dstc_v.at[b]], ssem[b],
                             add=True)

        def wait_scatter(b):
            pltpu.make_async_copy(rows[b], acc_sh.at[dstc_v.at[b]],
                                  ssem[b]).wait()

        def scale_rows(rows_v, wv_ref, n):
            @plsc.parallel_loop(0, n, 1, unroll=8,
                                carry=jnp.zeros((_L,), jnp.int32))
            def _(i, isplat):
                wv = plsc.load_gather(wv_ref, [isplat])
                for kk in range(_GD // _L):
                    sl = (i, pl.ds(kk * _L, _L))
                    rows_v[sl] = rows_v[sl] * wv
                return isplat + 1

        def process(b):
            scale_rows(rows[b], wcb_v.at[b], chunk)

        _ring(_D, nfull, start_fetch, wait_fetch, process, start_scatter,
              wait_scatter)

        if tail:
            off = base_e + main
            pltpu.sync_copy(src_hbm.at[pl.ds(off, tail)], srct_v)
            pltpu.sync_copy(dst_hbm.at[pl.ds(off, tail)], dstt_v)
            pltpu.sync_copy(w_hbm.at[pl.ds(off, tail)],
                            wcb_v.at[0, pl.ds(0, tail)])
            pltpu.sync_copy(x_hbm.at[srct_v], rows[0].at[pl.ds(0, tail)])
            scale_rows(rows[0], wcb_v.at[0], tail)
            pltpu.sync_copy(rows[0].at[pl.ds(0, tail)], acc_sh.at[dstt_v],
                            add=True)

        plsc.subcore_barrier()
        base_r = s * _RPS
        pltpu.sync_copy(acc_sh.at[pl.ds(base_r, _RPS)],
                        out_hbm.at[c, pl.ds(base_r, _RPS)])

    return k(src, dst, w, x)


def _sc_layer2(src, dst, hlo, hhi):
    chunk = 56
    eps = _E // _NS
    nfull = eps // chunk
    main = nfull * chunk
    tail = eps - main

    @functools.partial(
        pl.kernel,
        out_type=jax.ShapeDtypeStruct((_NC, _NPAD, _GD), jnp.float32),
        mesh=_mesh,
        compiler_params=_sc_params,
        scratch_types=[
            pltpu.VMEM((main,), jnp.int32),
            pltpu.VMEM((_D2, chunk), jnp.int32),
            pltpu.VMEM((tail,), jnp.int32),
            pltpu.VMEM((tail,), jnp.int32),
        ] + [pltpu.VMEM((chunk, _GD), jnp.float32)] * _D2
          + [pltpu.VMEM_SHARED((_NPAD, _GD), jnp.float32)]
          + [pltpu.SemaphoreType.DMA] * (2 * _D2),
    )
    def k(src_hbm, dst_hbm, hlo_hbm, hhi_hbm, out_hbm,
          srcb_v, dstc_v, srct_v, dstt_v, *rest):
        rows = rest[:_D2]
        acc_sh = rest[_D2]
        gsem = rest[_D2 + 1:_D2 + 1 + _D2]
        ssem = rest[_D2 + 1 + _D2:]
        c = lax.axis_index("c")
        s = lax.axis_index("s")
        base_e = s * eps
        srcb_cp = pltpu.async_copy(src_hbm.at[pl.ds(base_e, main)], srcb_v,
                                   gsem[0])
        _zero_acc(rows[0], acc_sh, s)
        srcb_cp.wait()
        plsc.subcore_barrier()

        def run(table):
            def start_fetch(jj, b):
                off = base_e + jj * chunk
                pltpu.async_copy(dst_hbm.at[pl.ds(off, chunk)],
                                 dstc_v.at[b], gsem[b])
                pltpu.async_copy(
                    table.at[srcb_v.at[pl.ds(jj * chunk, chunk)]],
                    rows[b], gsem[b])

            def wait_fetch(jj, b):
                off = base_e + jj * chunk
                pltpu.make_async_copy(dst_hbm.at[pl.ds(off, chunk)],
                                      dstc_v.at[b], gsem[b]).wait()
                pltpu.make_async_copy(
                    table.at[srcb_v.at[pl.ds(jj * chunk, chunk)]],
                    rows[b], gsem[b]).wait()

            def start_scatter(b):
                pltpu.async_copy(rows[b], acc_sh.at[dstc_v.at[b]], ssem[b],
                                 add=True)

            def wait_scatter(b):
                pltpu.make_async_copy(rows[b], acc_sh.at[dstc_v.at[b]],
                                      ssem[b]).wait()

            _ring(_D2, nfull, start_fetch, wait_fetch, lambda b: None,
                  start_scatter, wait_scatter)

            if tail:
                off = base_e + main
                pltpu.sync_copy(src_hbm.at[pl.ds(off, tail)], srct_v)
                pltpu.sync_copy(dst_hbm.at[pl.ds(off, tail)], dstt_v)
                pltpu.sync_copy(table.at[srct_v], rows[0].at[pl.ds(0, tail)])
                pltpu.sync_copy(rows[0].at[pl.ds(0, tail)], acc_sh.at[dstt_v],
                                add=True)

        @pl.when(c == 0)
        def _():
            run(hlo_hbm)

        @pl.when(c == 1)
        def _():
            run(hhi_hbm)

        plsc.subcore_barrier()
        base_r = s * _RPS
        pltpu.sync_copy(acc_sh.at[pl.ds(base_r, _RPS)],
                        out_hbm.at[c, pl.ds(base_r, _RPS)])

    return k(src, dst, hlo, hhi)


_BN = 1000


def _tc_h1(agg1, x, W_rel, W_root, b):
    def body(a_ref, x_ref, wrel_ref, wroot_ref, b_ref, o_ref, o2_ref):
        aggsum = a_ref[0, :, :] + a_ref[1, :, :]
        r = jnp.dot(aggsum, wrel_ref[...], preferred_element_type=jnp.float32)
        r = r + jnp.dot(x_ref[...], wroot_ref[...],
                        preferred_element_type=jnp.float32) + b_ref[...]
        o_ref[...] = r[:, :_GD]
        o2_ref[...] = r[:, _GD:]

    return pl.pallas_call(
        body,
        grid=(_N // _BN,),
        in_specs=[
            pl.BlockSpec((_NC, _BN, _GD), lambda i: (0, i, 0)),
            pl.BlockSpec((_BN, _GD), lambda i: (i, 0)),
            pl.BlockSpec((_GD, _H1), lambda i: (0, 0)),
            pl.BlockSpec((_GD, _H1), lambda i: (0, 0)),
            pl.BlockSpec((1, _H1), lambda i: (0, 0)),
        ],
        out_specs=[pl.BlockSpec((_BN, _GD), lambda i: (i, 0)),
                   pl.BlockSpec((_BN, _GD), lambda i: (i, 0))],
        out_shape=[jax.ShapeDtypeStruct((_N, _GD), jnp.float32),
                   jax.ShapeDtypeStruct((_N, _GD), jnp.float32)],
    )(agg1, x, W_rel, W_root, b)


def _tc_out(agg2, hlo, hhi, W_rel, W_root, b):
    def body(a_ref, hlo_ref, hhi_ref, wrel_ref, wroot_ref, b_ref, o_ref):
        r = jnp.dot(a_ref[0, :, :], wrel_ref[:_GD, :],
                    preferred_element_type=jnp.float32)
        r = r + jnp.dot(a_ref[1, :, :], wrel_ref[_GD:, :],
                        preferred_element_type=jnp.float32)
        r = r + jnp.dot(hlo_ref[...], wroot_ref[:_GD, :],
                        preferred_element_type=jnp.float32)
        r = r + jnp.dot(hhi_ref[...], wroot_ref[_GD:, :],
                        preferred_element_type=jnp.float32)
        o_ref[...] = r + b_ref[...]

    return pl.pallas_call(
        body,
        grid=(_N // _BN,),
        in_specs=[
            pl.BlockSpec((_NC, _BN, _GD), lambda i: (0, i, 0)),
            pl.BlockSpec((_BN, _GD), lambda i: (i, 0)),
            pl.BlockSpec((_BN, _GD), lambda i: (i, 0)),
            pl.BlockSpec((_H1, _H2), lambda i: (0, 0)),
            pl.BlockSpec((_H1, _H2), lambda i: (0, 0)),
            pl.BlockSpec((1, _H2), lambda i: (0, 0)),
        ],
        out_specs=pl.BlockSpec((_BN, _H2), lambda i: (i, 0)),
        out_shape=jax.ShapeDtypeStruct((_N, _H2), jnp.float32),
    )(agg2, hlo, hhi, W_rel, W_root, b)


def kernel(node_features, edge_index, edge_type, W1_rel, b1_rel, W1_root,
           W2_rel, b2_rel, W2_root):
    src = edge_index[0]
    dst = edge_index[1]
    b1 = b1_rel.reshape(1, _H1)
    b2 = b2_rel.reshape(1, _H2)

    agg1 = _sc_layer1(src, dst, edge_type, node_features)
    hlo, hhi = _tc_h1(agg1, node_features, W1_rel, W1_root, b1)
    agg2 = _sc_layer2(src, dst, hlo, hhi)
    out = _tc_out(agg2, hlo, hhi, W2_rel, W2_root, b2)
    return out
